# Initial kernel scaffold; baseline (speedup 1.0000x reference)
import jax
import jax.numpy as jnp
from jax.experimental import pallas as pl


def kernel(token_ids, mask):
    raise NotImplementedError("write your pallas kernel here")


# speedup vs baseline: 1.7120x; 1.7120x over previous
import functools

import jax
import jax.numpy as jnp
from jax import lax
from jax.experimental import pallas as pl
from jax.experimental.pallas import tpu as pltpu
from jax.experimental.pallas import tpu_sc as plsc

VOCAB = 100000
BATCH = 1024
SEQ = 200
LANES = 16
NUM_CORES = 2
NUM_SUBCORES = 16
NUM_WORKERS = NUM_CORES * NUM_SUBCORES
ROWS_PER_WORKER = BATCH // NUM_WORKERS
SEQ_PAD = 208
CHUNKS = SEQ_PAD // LANES


def _count_body(tok_hbm, val_hbm, out_hbm, tok_v, val_v, row_buf):
    wid = lax.axis_index("s") * NUM_CORES + lax.axis_index("c")
    base = wid * ROWS_PER_WORKER

    zeros16 = jnp.zeros((LANES,), jnp.float32)

    def zero_body(i, carry):
        row_buf[pl.ds(i * LANES, LANES)] = zeros16
        return carry

    lax.fori_loop(0, VOCAB // LANES, zero_body, 0)

    def row_body(r, carry):
        row = base + r
        pltpu.sync_copy(tok_hbm.at[row], tok_v)
        pltpu.sync_copy(val_hbm.at[row], val_v)
        for c in range(CHUNKS):
            idx = tok_v[pl.ds(c * LANES, LANES)]
            v = val_v[pl.ds(c * LANES, LANES)]
            plsc.addupdate_scatter(row_buf, [idx], v)
        pltpu.sync_copy(row_buf, out_hbm.at[row])
        for c in range(CHUNKS):
            idx = tok_v[pl.ds(c * LANES, LANES)]
            plsc.store_scatter(row_buf, [idx], zeros16)
        return carry

    lax.fori_loop(0, ROWS_PER_WORKER, row_body, 0)


_count_kernel = functools.partial(
    pl.kernel,
    out_type=jax.ShapeDtypeStruct((BATCH, VOCAB), jnp.float32),
    mesh=plsc.VectorSubcoreMesh(core_axis_name="c", subcore_axis_name="s"),
    scratch_types=[
        pltpu.VMEM((SEQ_PAD,), jnp.int32),
        pltpu.VMEM((SEQ_PAD,), jnp.float32),
        pltpu.VMEM((VOCAB,), jnp.float32),
    ],
    compiler_params=pltpu.CompilerParams(needs_layout_passes=False),
)(_count_body)


@jax.jit
def kernel(token_ids, mask):
    tok = jnp.pad(token_ids.astype(jnp.int32), ((0, 0), (0, SEQ_PAD - SEQ)))
    val = jnp.pad(mask.astype(jnp.float32), ((0, 0), (0, SEQ_PAD - SEQ)))
    return _count_kernel(tok, val)
